# Initial kernel scaffold; baseline (speedup 1.0000x reference)
import jax
import jax.numpy as jnp
from jax.experimental import pallas as pl


def kernel(x_base, x_joint, x_foot, edge_index_bj, edge_index_jf, edge_index_fb, params):
    raise NotImplementedError("write your pallas kernel here")


# speedup vs baseline: 4.1948x; 4.1948x over previous
import functools

import jax
import jax.numpy as jnp
from jax import lax
from jax.experimental import pallas as pl
from jax.experimental.pallas import tpu as pltpu
from jax.experimental.pallas import tpu_sc as plsc

F32 = jnp.float32

NC = 2
NS = 16
NW = NC * NS

CHUNK = 128
HCOLS = 80


def _linear(x, w, b, relu):
    n, d = x.shape
    dout = w.shape[1]
    blk = 1000

    def body(x_ref, w_ref, b_ref, o_ref):
        y = jnp.dot(x_ref[...], w_ref[...], preferred_element_type=F32)
        y = y + b_ref[...]
        if relu:
            y = jnp.maximum(y, 0.0)
        o_ref[...] = y

    return pl.pallas_call(
        body,
        grid=(n // blk,),
        in_specs=[
            pl.BlockSpec((blk, d), lambda i: (i, 0)),
            pl.BlockSpec((d, dout), lambda i: (0, 0)),
            pl.BlockSpec((1, dout), lambda i: (0, 0)),
        ],
        out_specs=pl.BlockSpec((blk, dout), lambda i: (i, 0)),
        out_shape=jax.ShapeDtypeStruct((n, dout), F32),
    )(x, w, b.reshape(1, dout))


def _edge_scores(hs_g3, hd_g3, att):
    eb = hs_g3.shape[0]
    rb = 10
    g = eb // rb

    def body(hs_ref, hd_ref, att_ref, e_ref, bm_ref):
        m = hs_ref[...] + hd_ref[...]
        m = jnp.where(m > 0, m, 0.2 * m)
        a = att_ref[0, :]
        e = lax.dot_general(m, a, (((2,), (0,)), ((), ())),
                            preferred_element_type=F32)
        e_ref[...] = e[None]
        bm_ref[...] = jnp.full((1, 1, 128), jnp.max(e), F32)

    return pl.pallas_call(
        body,
        grid=(g,),
        in_specs=[
            pl.BlockSpec((rb, 128, 128), lambda i: (i, 0, 0)),
            pl.BlockSpec((rb, 128, 128), lambda i: (i, 0, 0)),
            pl.BlockSpec((1, 128), lambda i: (0, 0)),
        ],
        out_specs=[
            pl.BlockSpec((1, rb, 128), lambda i: (i, 0, 0)),
            pl.BlockSpec((1, 1, 128), lambda i: (i, 0, 0)),
        ],
        out_shape=[
            jax.ShapeDtypeStruct((g, rb, 128), F32),
            jax.ShapeDtypeStruct((g, 1, 128), F32),
        ],
    )(hs_g3, hd_g3, att.reshape(1, 128))


def _edge_weights(e2, bmax, hs_g3):
    eb = hs_g3.shape[0]
    rb = 10
    g = eb // rb

    def body(e_ref, bm_ref, hs_ref, wa_ref):
        mglob = jnp.max(bm_ref[...])
        ex = jnp.exp(e_ref[0] - mglob)
        w = hs_ref[...] * ex[:, :, None]
        i32_ = lax.broadcasted_iota(jnp.int32, (rb, 128, 32), 2)
        x32 = jnp.where(i32_ == 0, ex[:, :, None], 0.0)
        wa_ref[0] = w[:, :, :HCOLS]
        wa_ref[1] = jnp.concatenate([w[:, :, HCOLS:], x32], axis=2)

    return pl.pallas_call(
        body,
        grid=(g,),
        in_specs=[
            pl.BlockSpec((1, rb, 128), lambda i: (i, 0, 0)),
            pl.BlockSpec((g, 1, 128), lambda i: (0, 0, 0)),
            pl.BlockSpec((rb, 128, 128), lambda i: (i, 0, 0)),
        ],
        out_specs=pl.BlockSpec((2, rb, 128, HCOLS), lambda i: (0, i, 0, 0)),
        out_shape=jax.ShapeDtypeStruct((2, eb, 128, HCOLS), F32),
    )(e2, bmax, hs_g3)


def _finalize(p, bias, n):
    blk = 1000

    def body(p_ref, b_ref, o_ref):
        lo = p_ref[0]
        hi = p_ref[1]
        u = jnp.concatenate([lo, hi[:, :128 - HCOLS]], axis=1)
        den = hi[:, 128 - HCOLS:129 - HCOLS]
        y = u / (den + 1e-16) + b_ref[...]
        o_ref[...] = jnp.maximum(y, 0.0)

    return pl.pallas_call(
        body,
        grid=(n // blk,),
        in_specs=[
            pl.BlockSpec((2, blk, HCOLS), lambda i: (0, i, 0)),
            pl.BlockSpec((1, 128), lambda i: (0, 0)),
        ],
        out_specs=pl.BlockSpec((blk, 128), lambda i: (i, 0)),
        out_shape=jax.ShapeDtypeStruct((n, 128), F32),
    )(p, bias.reshape(1, 128))


def _sc_gather2(hs, hd, src, dst):
    n, h = hs.shape
    e = src.shape[0]
    nchunks = e // CHUNK
    mesh = plsc.VectorSubcoreMesh(core_axis_name="c", subcore_axis_name="s")

    @functools.partial(
        pl.kernel,
        out_type=(jax.ShapeDtypeStruct((e, h), F32),
                  jax.ShapeDtypeStruct((e, h), F32)),
        mesh=mesh,
        scratch_types=[
            pltpu.VMEM((CHUNK,), jnp.int32),
            pltpu.VMEM((CHUNK,), jnp.int32),
            pltpu.VMEM((CHUNK, h), F32),
            pltpu.VMEM((CHUNK, h), F32),
            pltpu.SemaphoreType.DMA,
            pltpu.SemaphoreType.DMA,
        ],
    )
    def k(hs_hbm, hd_hbm, src_hbm, dst_hbm, ohs_hbm, ohd_hbm,
          si, di, hsb, hdb, s1, s2):
        wid = lax.axis_index("s") * NC + lax.axis_index("c")

        def step(j, carry):
            off = (wid + j * NW) * CHUNK
            pltpu.sync_copy(src_hbm.at[pl.ds(off, CHUNK)], si)
            pltpu.sync_copy(dst_hbm.at[pl.ds(off, CHUNK)], di)
            c1 = pltpu.async_copy(hs_hbm.at[si], hsb, s1)
            c2 = pltpu.async_copy(hd_hbm.at[di], hdb, s2)
            c1.wait()
            c2.wait()
            pltpu.sync_copy(hsb, ohs_hbm.at[pl.ds(off, CHUNK)])
            pltpu.sync_copy(hdb, ohd_hbm.at[pl.ds(off, CHUNK)])
            return carry

        nj = (nchunks - wid + NW - 1) // NW
        lax.fori_loop(0, nj, step, 0)

    return k(hs, hd, src, dst)


def _sc_scatter(wa2, dst, zeros):
    e = dst.shape[0]
    n = zeros.shape[0]
    nchunks = e // CHUNK
    rpt = n // NS
    mesh = plsc.VectorSubcoreMesh(core_axis_name="c", subcore_axis_name="s")

    @functools.partial(
        pl.kernel,
        out_type=jax.ShapeDtypeStruct((NC, n, HCOLS), F32),
        mesh=mesh,
        scratch_types=[
            pltpu.VMEM((CHUNK,), jnp.int32),
            pltpu.VMEM((CHUNK, HCOLS), F32),
            pltpu.VMEM((rpt, HCOLS), F32),
            pltpu.VMEM_SHARED((n, HCOLS), F32),
        ],
        compiler_params=pltpu.CompilerParams(use_tc_tiling_on_sc=False),
    )
    def k(wa_hbm, dst_hbm, z_hbm, out_hbm, di, rows, bounce, acc):
        c = lax.axis_index("c")
        s = lax.axis_index("s")
        r0 = s * rpt

        pltpu.sync_copy(z_hbm.at[pl.ds(r0, rpt)], bounce)
        pltpu.sync_copy(bounce, acc.at[pl.ds(r0, rpt)])
        plsc.subcore_barrier()

        def step(j, carry):
            off = (s + j * NS) * CHUNK
            pltpu.sync_copy(dst_hbm.at[pl.ds(off, CHUNK)], di)
            pltpu.sync_copy(wa_hbm.at[c, pl.ds(off, CHUNK)], rows)
            pltpu.sync_copy(rows, acc.at[di], add=True)
            return carry

        nj = (nchunks - s + NS - 1) // NS
        lax.fori_loop(0, nj, step, 0)
        plsc.subcore_barrier()

        pltpu.sync_copy(acc.at[pl.ds(r0, rpt)], bounce)
        pltpu.sync_copy(bounce, out_hbm.at[c, pl.ds(r0, rpt)])

    return k(wa2, dst, zeros)


def kernel(x_base, x_joint, x_foot, edge_index_bj, edge_index_jf,
           edge_index_fb, params):
    n = x_base.shape[0]
    e = edge_index_bj.shape[1]
    eb = e // 128

    npad = ((n + 8 * NS - 1) // (8 * NS)) * (8 * NS)
    zeros_acc = jnp.zeros((npad, HCOLS), F32)

    def gat(h_src, h_dst, edges, pre):
        src = edges[0]
        dst = edges[1]
        hs = _linear(h_src, params[pre + "_W_l"], params[pre + "_b_l"], False)
        hd = _linear(h_dst, params[pre + "_W_r"], params[pre + "_b_r"], False)
        hs_g, hd_g = _sc_gather2(hs, hd, src, dst)
        hs_g3 = hs_g.reshape(eb, 128, 128)
        hd_g3 = hd_g.reshape(eb, 128, 128)
        e2, bmax = _edge_scores(hs_g3, hd_g3, params[pre + "_att"])
        wa = _edge_weights(e2, bmax, hs_g3)
        p = _sc_scatter(wa.reshape(2, e, HCOLS), dst, zeros_acc)
        return _finalize(p, params[pre + "_bias"], n)

    hb = _linear(x_base, params["enc_W_base"], params["enc_b_base"], True)
    hj = _linear(x_joint, params["enc_W_joint"], params["enc_b_joint"], True)
    hf = _linear(x_foot, params["enc_W_foot"], params["enc_b_foot"], True)

    hj1 = gat(hb, hj, edge_index_bj, "bj0")
    hf1 = gat(hj, hf, edge_index_jf, "jf0")
    hf2 = gat(hj1, hf1, edge_index_jf, "jf1")

    out_dim = params["dec_W"].shape[1]
    dec_w = jnp.pad(params["dec_W"], ((0, 0), (0, 128 - out_dim)))
    dec_b = jnp.pad(params["dec_b"], (0, 128 - out_dim))
    out = _linear(hf2, dec_w, dec_b, False)
    return out[:, :out_dim]
